# baseline (device time: 163286 ns/iter reference)
import jax
import jax.numpy as jnp
from jax import lax
from jax.experimental import pallas as pl
from jax.experimental.pallas import tpu as pltpu

N_DEV = 4


def kernel(A, B):
    m, k_shard = A.shape
    _, n = B.shape
    m_chunk = m // N_DEV

    def body(a_ref, b_ref, out_ref, rs_ref, bbf_ref,
             rs_send_sems, rs_recv_sems, ag_send_sems, ag_recv_sems):
        my = lax.axis_index("i")
        left = (my - 1) % N_DEV
        right = (my + 1) % N_DEV

        barrier_sem = pltpu.get_barrier_semaphore()
        for nbr in (left, right):
            pl.semaphore_signal(
                barrier_sem, inc=1,
                device_id=(nbr,), device_id_type=pl.DeviceIdType.MESH,
            )
        pl.semaphore_wait(barrier_sem, 2)

        bbf_ref[...] = b_ref[...].astype(jnp.bfloat16)

        def partial(c):
            a = a_ref[pl.ds(c * m_chunk, m_chunk), :].astype(jnp.bfloat16)
            return jnp.dot(a, bbf_ref[...], preferred_element_type=jnp.float32)

        rs_ref[0] = partial((my - 1) % N_DEV).astype(jnp.bfloat16)
        for h in range(N_DEV - 1):
            rdma = pltpu.make_async_remote_copy(
                src_ref=rs_ref.at[h],
                dst_ref=rs_ref.at[h + 1],
                send_sem=rs_send_sems.at[h],
                recv_sem=rs_recv_sems.at[h],
                device_id=(right,),
                device_id_type=pl.DeviceIdType.MESH,
            )
            rdma.start()
            c = (my - h - 2) % N_DEV
            p = partial(c)
            rdma.wait()
            acc = rs_ref[h + 1].astype(jnp.float32) + p
            if h < N_DEV - 2:
                rs_ref[h + 1] = acc.astype(jnp.bfloat16)
            else:
                z = acc
                silu = z * (1.0 / (1.0 + jnp.exp(-z)))
                out_ref[pl.ds(my * m_chunk, m_chunk), :] = silu.astype(
                    jnp.bfloat16)

        for h in range(N_DEV - 1):
            origin = (my - h) % N_DEV
            rows = pl.ds(origin * m_chunk, m_chunk)
            rdma = pltpu.make_async_remote_copy(
                src_ref=out_ref.at[rows, :],
                dst_ref=out_ref.at[rows, :],
                send_sem=ag_send_sems.at[h],
                recv_sem=ag_recv_sems.at[h],
                device_id=(right,),
                device_id_type=pl.DeviceIdType.MESH,
            )
            rdma.start()
            rdma.wait()

    return pl.pallas_call(
        body,
        out_shape=jax.ShapeDtypeStruct((m, n), jnp.bfloat16),
        in_specs=[
            pl.BlockSpec(memory_space=pltpu.VMEM),
            pl.BlockSpec(memory_space=pltpu.VMEM),
        ],
        out_specs=pl.BlockSpec(memory_space=pltpu.VMEM),
        scratch_shapes=[
            pltpu.VMEM((N_DEV, m_chunk, n), jnp.bfloat16),
            pltpu.VMEM((k_shard, n), jnp.bfloat16),
            pltpu.SemaphoreType.DMA((N_DEV - 1,)),
            pltpu.SemaphoreType.DMA((N_DEV - 1,)),
            pltpu.SemaphoreType.DMA((N_DEV - 1,)),
            pltpu.SemaphoreType.DMA((N_DEV - 1,)),
        ],
        compiler_params=pltpu.CompilerParams(collective_id=0),
    )(A, B)


# device time: 96010 ns/iter; 1.7007x vs baseline; 1.7007x over previous
import jax
import jax.numpy as jnp
from jax import lax
from jax.experimental import pallas as pl
from jax.experimental.pallas import tpu as pltpu

N_DEV = 4


def kernel(A, B):
    m, k_shard = A.shape
    _, n = B.shape
    m_chunk = m // N_DEV
    half = n // 2

    def body(a_ref, b_ref, out_ref, rsr_ref, rsl_ref, bbf_ref,
             rsr_send, rsr_recv, rsl_send, rsl_recv,
             agr_send, agr_recv, agl_send, agl_recv):
        my = lax.axis_index("i")
        left = (my - 1) % N_DEV
        right = (my + 1) % N_DEV

        barrier_sem = pltpu.get_barrier_semaphore()
        for nbr in (left, right):
            pl.semaphore_signal(
                barrier_sem, inc=1,
                device_id=(nbr,), device_id_type=pl.DeviceIdType.MESH,
            )
        pl.semaphore_wait(barrier_sem, 2)

        bbf_ref[...] = b_ref[...].astype(jnp.bfloat16)

        def partial_half(c, col0):
            a = a_ref[pl.ds(c * m_chunk, m_chunk), :].astype(jnp.bfloat16)
            b = bbf_ref[:, col0:col0 + half]
            return jnp.dot(a, b, preferred_element_type=jnp.float32)

        def silu(z):
            return z * (1.0 / (1.0 + jnp.exp(-z)))

        rsr_ref[0] = partial_half((my - 1) % N_DEV, 0).astype(jnp.bfloat16)
        rsl_ref[0] = partial_half((my + 1) % N_DEV, half).astype(jnp.bfloat16)
        for h in range(N_DEV - 1):
            rd_r = pltpu.make_async_remote_copy(
                src_ref=rsr_ref.at[h],
                dst_ref=rsr_ref.at[h + 1],
                send_sem=rsr_send.at[h],
                recv_sem=rsr_recv.at[h],
                device_id=(right,),
                device_id_type=pl.DeviceIdType.MESH,
            )
            rd_l = pltpu.make_async_remote_copy(
                src_ref=rsl_ref.at[h],
                dst_ref=rsl_ref.at[h + 1],
                send_sem=rsl_send.at[h],
                recv_sem=rsl_recv.at[h],
                device_id=(left,),
                device_id_type=pl.DeviceIdType.MESH,
            )
            rd_r.start()
            rd_l.start()
            p_r = partial_half((my - h - 2) % N_DEV, 0)
            p_l = partial_half((my + h + 2) % N_DEV, half)
            rd_r.wait()
            rd_l.wait()
            acc_r = rsr_ref[h + 1].astype(jnp.float32) + p_r
            acc_l = rsl_ref[h + 1].astype(jnp.float32) + p_l
            if h < N_DEV - 2:
                rsr_ref[h + 1] = acc_r.astype(jnp.bfloat16)
                rsl_ref[h + 1] = acc_l.astype(jnp.bfloat16)
            else:
                rows = pl.ds(my * m_chunk, m_chunk)
                out_ref[rows, 0:half] = silu(acc_r).astype(jnp.bfloat16)
                out_ref[rows, half:n] = silu(acc_l).astype(jnp.bfloat16)

        for h in range(N_DEV - 1):
            rows_r = pl.ds(((my - h) % N_DEV) * m_chunk, m_chunk)
            rows_l = pl.ds(((my + h) % N_DEV) * m_chunk, m_chunk)
            rd_r = pltpu.make_async_remote_copy(
                src_ref=out_ref.at[rows_r, 0:half],
                dst_ref=out_ref.at[rows_r, 0:half],
                send_sem=agr_send.at[h],
                recv_sem=agr_recv.at[h],
                device_id=(right,),
                device_id_type=pl.DeviceIdType.MESH,
            )
            rd_l = pltpu.make_async_remote_copy(
                src_ref=out_ref.at[rows_l, half:n],
                dst_ref=out_ref.at[rows_l, half:n],
                send_sem=agl_send.at[h],
                recv_sem=agl_recv.at[h],
                device_id=(left,),
                device_id_type=pl.DeviceIdType.MESH,
            )
            rd_r.start()
            rd_l.start()
            rd_r.wait()
            rd_l.wait()

    return pl.pallas_call(
        body,
        out_shape=jax.ShapeDtypeStruct((m, n), jnp.bfloat16),
        in_specs=[
            pl.BlockSpec(memory_space=pltpu.VMEM),
            pl.BlockSpec(memory_space=pltpu.VMEM),
        ],
        out_specs=pl.BlockSpec(memory_space=pltpu.VMEM),
        scratch_shapes=[
            pltpu.VMEM((N_DEV, m_chunk, half), jnp.bfloat16),
            pltpu.VMEM((N_DEV, m_chunk, half), jnp.bfloat16),
            pltpu.VMEM((k_shard, n), jnp.bfloat16),
            pltpu.SemaphoreType.DMA((N_DEV - 1,)),
            pltpu.SemaphoreType.DMA((N_DEV - 1,)),
            pltpu.SemaphoreType.DMA((N_DEV - 1,)),
            pltpu.SemaphoreType.DMA((N_DEV - 1,)),
            pltpu.SemaphoreType.DMA((N_DEV - 1,)),
            pltpu.SemaphoreType.DMA((N_DEV - 1,)),
            pltpu.SemaphoreType.DMA((N_DEV - 1,)),
            pltpu.SemaphoreType.DMA((N_DEV - 1,)),
        ],
        compiler_params=pltpu.CompilerParams(collective_id=0),
    )(A, B)


# device time: 84518 ns/iter; 1.9320x vs baseline; 1.1360x over previous
import jax
import jax.numpy as jnp
from jax import lax
from jax.experimental import pallas as pl
from jax.experimental.pallas import tpu as pltpu

N_DEV = 4
N_CHAIN = 4


def kernel(A, B):
    m, k_shard = A.shape
    _, n = B.shape
    m_chunk = m // N_DEV
    q = n // N_CHAIN

    SIGMA = (1, 1, -1, -1)

    def body(a_ref, b_ref, out_ref, rs0, rs1, rs2, rs3, bbf_ref,
             rs_send, rs_recv, ag_send, ag_recv):
        rs_refs = (rs0, rs1, rs2, rs3)
        my = lax.axis_index("i")
        left = (my - 1) % N_DEV
        right = (my + 1) % N_DEV
        tgt = tuple((my + s) % N_DEV for s in SIGMA)

        barrier_sem = pltpu.get_barrier_semaphore()
        for nbr in (left, right):
            pl.semaphore_signal(
                barrier_sem, inc=1,
                device_id=(nbr,), device_id_type=pl.DeviceIdType.MESH,
            )
        pl.semaphore_wait(barrier_sem, 2)

        bbf_ref[...] = b_ref[...].astype(jnp.bfloat16)

        def partial_q(c, ci):
            a = a_ref[pl.ds(c * m_chunk, m_chunk), :].astype(jnp.bfloat16)
            b = bbf_ref[:, ci * q:(ci + 1) * q]
            return jnp.dot(a, b, preferred_element_type=jnp.float32)

        def silu(z):
            return z * (1.0 / (1.0 + jnp.exp(-z)))

        def rs_start(ci, h):
            rdma = pltpu.make_async_remote_copy(
                src_ref=rs_refs[ci].at[h],
                dst_ref=rs_refs[ci].at[h + 1],
                send_sem=rs_send.at[ci, h],
                recv_sem=rs_recv.at[ci, h],
                device_id=(tgt[ci],),
                device_id_type=pl.DeviceIdType.MESH,
            )
            rdma.start()
            return rdma

        def ag_start(ci, h):
            origin = (my - SIGMA[ci] * h) % N_DEV
            rows = pl.ds(origin * m_chunk, m_chunk)
            rdma = pltpu.make_async_remote_copy(
                src_ref=out_ref.at[rows, ci * q:(ci + 1) * q],
                dst_ref=out_ref.at[rows, ci * q:(ci + 1) * q],
                send_sem=ag_send.at[ci, h],
                recv_sem=ag_recv.at[ci, h],
                device_id=(tgt[ci],),
                device_id_type=pl.DeviceIdType.MESH,
            )
            rdma.start()
            return rdma

        inflight = {}
        for group in ((0, 2), (1, 3)):
            for ci in group:
                c0 = (my - SIGMA[ci]) % N_DEV
                rs_refs[ci][0] = partial_q(c0, ci).astype(jnp.bfloat16)
            for ci in group:
                inflight[(ci, "rs", 0)] = rs_start(ci, 0)

        for h in range(N_DEV - 1):
            for group in ((0, 2), (1, 3)):
                ps = {ci: partial_q((my - SIGMA[ci] * (h + 2)) % N_DEV, ci)
                      for ci in group}
                for ci in group:
                    inflight.pop((ci, "rs", h)).wait()
                    acc = rs_refs[ci][h + 1].astype(jnp.float32) + ps[ci]
                    if h < N_DEV - 2:
                        rs_refs[ci][h + 1] = acc.astype(jnp.bfloat16)
                        inflight[(ci, "rs", h + 1)] = rs_start(ci, h + 1)
                    else:
                        rows = pl.ds(my * m_chunk, m_chunk)
                        out_ref[rows, ci * q:(ci + 1) * q] = silu(acc).astype(
                            jnp.bfloat16)
                        inflight[(ci, "ag", 0)] = ag_start(ci, 0)

        for h in range(N_DEV - 1):
            for group in ((0, 2), (1, 3)):
                for ci in group:
                    inflight.pop((ci, "ag", h)).wait()
                    if h < N_DEV - 2:
                        inflight[(ci, "ag", h + 1)] = ag_start(ci, h + 1)

    return pl.pallas_call(
        body,
        out_shape=jax.ShapeDtypeStruct((m, n), jnp.bfloat16),
        in_specs=[
            pl.BlockSpec(memory_space=pltpu.VMEM),
            pl.BlockSpec(memory_space=pltpu.VMEM),
        ],
        out_specs=pl.BlockSpec(memory_space=pltpu.VMEM),
        scratch_shapes=[
            pltpu.VMEM((N_DEV, m_chunk, q), jnp.bfloat16),
            pltpu.VMEM((N_DEV, m_chunk, q), jnp.bfloat16),
            pltpu.VMEM((N_DEV, m_chunk, q), jnp.bfloat16),
            pltpu.VMEM((N_DEV, m_chunk, q), jnp.bfloat16),
            pltpu.VMEM((k_shard, n), jnp.bfloat16),
            pltpu.SemaphoreType.DMA((N_CHAIN, N_DEV - 1)),
            pltpu.SemaphoreType.DMA((N_CHAIN, N_DEV - 1)),
            pltpu.SemaphoreType.DMA((N_CHAIN, N_DEV - 1)),
            pltpu.SemaphoreType.DMA((N_CHAIN, N_DEV - 1)),
        ],
        compiler_params=pltpu.CompilerParams(collective_id=0),
    )(A, B)


# device time: 84512 ns/iter; 1.9321x vs baseline; 1.0001x over previous
import jax
import jax.numpy as jnp
from jax import lax
from jax.experimental import pallas as pl
from jax.experimental.pallas import tpu as pltpu

N_DEV = 4
N_CHAIN = 4


def kernel(A, B):
    m, k_shard = A.shape
    _, n = B.shape
    m_chunk = m // N_DEV
    q = n // N_CHAIN

    SIGMA = (1, 1, -1, -1)

    def body(a_ref, b_ref, out_ref, rs0, rs1, rs2, rs3, bbf_ref,
             rs_send, rs_recv, ag_send, ag_recv):
        rs_refs = (rs0, rs1, rs2, rs3)
        my = lax.axis_index("i")
        left = (my - 1) % N_DEV
        right = (my + 1) % N_DEV
        tgt = tuple((my + s) % N_DEV for s in SIGMA)

        barrier_sem = pltpu.get_barrier_semaphore()
        for nbr in (left, right):
            pl.semaphore_signal(
                barrier_sem, inc=1,
                device_id=(nbr,), device_id_type=pl.DeviceIdType.MESH,
            )
        pl.semaphore_wait(barrier_sem, 2)

        bbf_ref[...] = b_ref[...].astype(jnp.bfloat16)

        def partial_q(c, ci):
            a = a_ref[pl.ds(c * m_chunk, m_chunk), :].astype(jnp.bfloat16)
            b = bbf_ref[:, ci * q:(ci + 1) * q]
            return jnp.dot(a, b, preferred_element_type=jnp.float32)

        def silu(z):
            return z * (1.0 / (1.0 + jnp.exp(-z)))

        def rs_start(ci, h):
            rdma = pltpu.make_async_remote_copy(
                src_ref=rs_refs[ci].at[h],
                dst_ref=rs_refs[ci].at[h + 1],
                send_sem=rs_send.at[ci, h],
                recv_sem=rs_recv.at[ci, h],
                device_id=(tgt[ci],),
                device_id_type=pl.DeviceIdType.MESH,
            )
            rdma.start()
            return rdma

        def ag_start(ci, h):
            origin = (my - SIGMA[ci] * h) % N_DEV
            rows = pl.ds(origin * m_chunk, m_chunk)
            rdma = pltpu.make_async_remote_copy(
                src_ref=out_ref.at[rows, ci * q:(ci + 1) * q],
                dst_ref=out_ref.at[rows, ci * q:(ci + 1) * q],
                send_sem=ag_send.at[ci, h],
                recv_sem=ag_recv.at[ci, h],
                device_id=(tgt[ci],),
                device_id_type=pl.DeviceIdType.MESH,
            )
            rdma.start()
            return rdma

        inflight = {}
        done = []
        for group in ((0, 2), (1, 3)):
            for ci in group:
                c0 = (my - SIGMA[ci]) % N_DEV
                rs_refs[ci][0] = partial_q(c0, ci).astype(jnp.bfloat16)
            for ci in group:
                inflight[(ci, "rs", 0)] = rs_start(ci, 0)

        ps = {ci: partial_q((my - SIGMA[ci] * 2) % N_DEV, ci)
              for ci in (0, 2, 1, 3)}
        for h in range(N_DEV - 1):
            for group in ((0, 2), (1, 3)):
                for ci in group:
                    rdma = inflight.pop((ci, "rs", h))
                    rdma.wait_recv()
                    done.append(rdma)
                    acc = rs_refs[ci][h + 1].astype(jnp.float32) + ps[ci]
                    if h < N_DEV - 2:
                        rs_refs[ci][h + 1] = acc.astype(jnp.bfloat16)
                        inflight[(ci, "rs", h + 1)] = rs_start(ci, h + 1)
                    else:
                        rows = pl.ds(my * m_chunk, m_chunk)
                        out_ref[rows, ci * q:(ci + 1) * q] = silu(acc).astype(
                            jnp.bfloat16)
                        inflight[(ci, "ag", 0)] = ag_start(ci, 0)
                if h < N_DEV - 2:
                    for ci in group:
                        ps[ci] = partial_q(
                            (my - SIGMA[ci] * (h + 3)) % N_DEV, ci)

        for h in range(N_DEV - 1):
            for group in ((0, 2), (1, 3)):
                for ci in group:
                    rdma = inflight.pop((ci, "ag", h))
                    rdma.wait_recv()
                    done.append(rdma)
                    if h < N_DEV - 2:
                        inflight[(ci, "ag", h + 1)] = ag_start(ci, h + 1)

        for rdma in done:
            rdma.wait_send()

    return pl.pallas_call(
        body,
        out_shape=jax.ShapeDtypeStruct((m, n), jnp.bfloat16),
        in_specs=[
            pl.BlockSpec(memory_space=pltpu.VMEM),
            pl.BlockSpec(memory_space=pltpu.VMEM),
        ],
        out_specs=pl.BlockSpec(memory_space=pltpu.VMEM),
        scratch_shapes=[
            pltpu.VMEM((N_DEV, m_chunk, q), jnp.bfloat16),
            pltpu.VMEM((N_DEV, m_chunk, q), jnp.bfloat16),
            pltpu.VMEM((N_DEV, m_chunk, q), jnp.bfloat16),
            pltpu.VMEM((N_DEV, m_chunk, q), jnp.bfloat16),
            pltpu.VMEM((k_shard, n), jnp.bfloat16),
            pltpu.SemaphoreType.DMA((N_CHAIN, N_DEV - 1)),
            pltpu.SemaphoreType.DMA((N_CHAIN, N_DEV - 1)),
            pltpu.SemaphoreType.DMA((N_CHAIN, N_DEV - 1)),
            pltpu.SemaphoreType.DMA((N_CHAIN, N_DEV - 1)),
        ],
        compiler_params=pltpu.CompilerParams(collective_id=0),
    )(A, B)


# device time: 84456 ns/iter; 1.9334x vs baseline; 1.0007x over previous
import jax
import jax.numpy as jnp
from jax import lax
from jax.experimental import pallas as pl
from jax.experimental.pallas import tpu as pltpu

N_DEV = 4
NSEG = 4
N_CHAIN = 2 * NSEG
PAIRS = tuple((s, s + NSEG) for s in range(NSEG))


def kernel(A, B):
    m, k_shard = A.shape
    _, n = B.shape
    m_chunk = m // N_DEV
    q = n // N_CHAIN

    SIGMA = tuple(1 if ci < NSEG else -1 for ci in range(N_CHAIN))

    def body(a_ref, b_ref, out_ref, rs_ref, bbf_ref,
             rs_send, rs_recv, ag_send, ag_recv):
        my = lax.axis_index("i")
        left = (my - 1) % N_DEV
        right = (my + 1) % N_DEV
        tgt = tuple((my + s) % N_DEV for s in SIGMA)

        barrier_sem = pltpu.get_barrier_semaphore()
        for nbr in (left, right):
            pl.semaphore_signal(
                barrier_sem, inc=1,
                device_id=(nbr,), device_id_type=pl.DeviceIdType.MESH,
            )
        pl.semaphore_wait(barrier_sem, 2)

        bbf_ref[...] = b_ref[...].astype(jnp.bfloat16)

        def partial_q(c, ci):
            a = a_ref[pl.ds(c * m_chunk, m_chunk), :].astype(jnp.bfloat16)
            b = bbf_ref[:, ci * q:(ci + 1) * q]
            return jnp.dot(a, b, preferred_element_type=jnp.float32)

        def silu(z):
            return z * (1.0 / (1.0 + jnp.exp(-z)))

        def rs_start(ci, h):
            rdma = pltpu.make_async_remote_copy(
                src_ref=rs_ref.at[ci, h],
                dst_ref=rs_ref.at[ci, h + 1],
                send_sem=rs_send.at[ci, h],
                recv_sem=rs_recv.at[ci, h],
                device_id=(tgt[ci],),
                device_id_type=pl.DeviceIdType.MESH,
            )
            rdma.start()
            return rdma

        def ag_start(ci, h):
            origin = (my - SIGMA[ci] * h) % N_DEV
            rows = pl.ds(origin * m_chunk, m_chunk)
            rdma = pltpu.make_async_remote_copy(
                src_ref=out_ref.at[rows, ci * q:(ci + 1) * q],
                dst_ref=out_ref.at[rows, ci * q:(ci + 1) * q],
                send_sem=ag_send.at[ci, h],
                recv_sem=ag_recv.at[ci, h],
                device_id=(tgt[ci],),
                device_id_type=pl.DeviceIdType.MESH,
            )
            rdma.start()
            return rdma

        inflight = {}
        done = []
        for pair in PAIRS:
            for ci in pair:
                c0 = (my - SIGMA[ci]) % N_DEV
                rs_ref[ci, 0] = partial_q(c0, ci).astype(jnp.bfloat16)
            for ci in pair:
                inflight[(ci, "rs", 0)] = rs_start(ci, 0)

        ps = {}
        for pair in PAIRS:
            for ci in pair:
                ps[ci] = partial_q((my - SIGMA[ci] * 2) % N_DEV, ci)
        for h in range(N_DEV - 1):
            for pair in PAIRS:
                for ci in pair:
                    rdma = inflight.pop((ci, "rs", h))
                    rdma.wait_recv()
                    done.append(rdma)
                    acc = rs_ref[ci, h + 1].astype(jnp.float32) + ps[ci]
                    if h < N_DEV - 2:
                        rs_ref[ci, h + 1] = acc.astype(jnp.bfloat16)
                        inflight[(ci, "rs", h + 1)] = rs_start(ci, h + 1)
                    else:
                        rows = pl.ds(my * m_chunk, m_chunk)
                        out_ref[rows, ci * q:(ci + 1) * q] = silu(acc).astype(
                            jnp.bfloat16)
                        inflight[(ci, "ag", 0)] = ag_start(ci, 0)
                if h < N_DEV - 2:
                    for ci in pair:
                        ps[ci] = partial_q(
                            (my - SIGMA[ci] * (h + 3)) % N_DEV, ci)

        for h in range(N_DEV - 1):
            for pair in PAIRS:
                for ci in pair:
                    rdma = inflight.pop((ci, "ag", h))
                    rdma.wait_recv()
                    done.append(rdma)
                    if h < N_DEV - 2:
                        inflight[(ci, "ag", h + 1)] = ag_start(ci, h + 1)

        for rdma in done:
            rdma.wait_send()

    return pl.pallas_call(
        body,
        out_shape=jax.ShapeDtypeStruct((m, n), jnp.bfloat16),
        in_specs=[
            pl.BlockSpec(memory_space=pltpu.VMEM),
            pl.BlockSpec(memory_space=pltpu.VMEM),
        ],
        out_specs=pl.BlockSpec(memory_space=pltpu.VMEM),
        scratch_shapes=[
            pltpu.VMEM((N_CHAIN, N_DEV, m_chunk, q), jnp.bfloat16),
            pltpu.VMEM((k_shard, n), jnp.bfloat16),
            pltpu.SemaphoreType.DMA((N_CHAIN, N_DEV - 1)),
            pltpu.SemaphoreType.DMA((N_CHAIN, N_DEV - 1)),
            pltpu.SemaphoreType.DMA((N_CHAIN, N_DEV - 1)),
            pltpu.SemaphoreType.DMA((N_CHAIN, N_DEV - 1)),
        ],
        compiler_params=pltpu.CompilerParams(collective_id=0),
    )(A, B)
